# baseline (speedup 1.0000x reference)
import functools

import jax
import jax.numpy as jnp
from jax import lax
from jax.experimental import pallas as pl
from jax.experimental.pallas import tpu as pltpu
from jax.experimental.pallas import tpu_sc as plsc

GRID_SIZE = 4
SPLINE_ORDER = 3
NBASIS = GRID_SIZE + SPLINE_ORDER
N = 10000
E = 320000

NC, NS = 2, 16
NW = NC * NS
CHUNK = 64
NCHUNKS = 5120
CHUNKS_PER_W = NCHUNKS // NW
EPAD = NCHUNKS * CHUNK
ROWS_PER_TILE = 632
NPAD = NS * ROWS_PER_TILE
BLK = 400
NBLK = N // BLK

_SC_MESH = dict(core_axis_name="c", subcore_axis_name="s",
                num_cores=NC, num_subcores=NS)


def _silu(x):
  return x * (1.0 / (1.0 + jnp.exp(-x)))


def _spline_bases(x):
  h = 2.0 / GRID_SIZE
  g = [(t - SPLINE_ORDER) * h - 1.0 for t in range(GRID_SIZE + 2 * SPLINE_ORDER + 1)]
  d = [x - g[t] for t in range(len(g))]
  b = [jnp.where((x >= g[t]) & (x < g[t + 1]), 1.0, 0.0).astype(x.dtype)
       for t in range(len(g) - 1)]
  for k in range(1, SPLINE_ORDER + 1):
    inv = 1.0 / (k * h)
    nb = []
    for t in range(len(b) - 1):
      left = (d[t] * inv) * b[t]
      right = (d[t + k + 1] * (-inv)) * b[t + 1]
      nb.append(left + right)
    b = nb
  return b


def _dinv_from_deg(deg_blk):
  degs = deg_blk[0] + deg_blk[1] + 1.0
  return lax.rsqrt(degs)[:, 0:1]


def _kan1_body(x_ref, deg_ref, wb1_ref, ws1_ref, wb2x_ref, ws2x_ref,
               hs1_ref, pre2_ref):
  xb = x_ref[:]
  dinv = _dinv_from_deg(deg_ref[:])
  sil = _silu(xb)
  bs = _spline_bases(xb)
  h1 = jnp.dot(sil, wb1_ref[:], preferred_element_type=jnp.float32)
  p2 = jnp.dot(sil, wb2x_ref[:], preferred_element_type=jnp.float32)
  for c in range(NBASIS):
    h1 = h1 + jnp.dot(bs[c], ws1_ref[c], preferred_element_type=jnp.float32)
    p2 = p2 + jnp.dot(bs[c], ws2x_ref[c], preferred_element_type=jnp.float32)
  hs1_ref[:] = h1 * dinv
  pre2_ref[:] = p2


def _kan2_body(hs1_ref, part_ref, pre2_ref, deg_ref, b1_ref, wb2a_ref,
               ws2a_ref, hs2a_ref, hs2b_ref, dinv_ref):
  degs = deg_ref[0] + deg_ref[1] + 1.0
  dinv16 = lax.rsqrt(degs)
  dinv = dinv16[:, 0:1]
  agg = dinv * (part_ref[0] + part_ref[1] + hs1_ref[:]) + b1_ref[:]
  sil = _silu(agg)
  bs = _spline_bases(agg)
  h2 = pre2_ref[:] + jnp.dot(sil, wb2a_ref[:], preferred_element_type=jnp.float32)
  for c in range(NBASIS):
    h2 = h2 + jnp.dot(bs[c], ws2a_ref[c], preferred_element_type=jnp.float32)
  hs2 = h2 * dinv
  hs2a_ref[:] = hs2[:, :32]
  hs2b_ref[:] = hs2[:, 32:]
  dinv_ref[:] = dinv16


def _row_spec(f):
  return pl.BlockSpec((BLK, f), lambda i: (i, 0))


def _part_spec(f):
  return pl.BlockSpec((2, BLK, f), lambda i: (0, i, 0))


def _full_spec(shape):
  nd = len(shape)
  return pl.BlockSpec(shape, lambda i, _nd=nd: (0,) * _nd)


def _kan1_call(x, degp, wb1, ws1, wb2x, ws2x):
  return pl.pallas_call(
      _kan1_body,
      grid=(NBLK,),
      in_specs=[
          _row_spec(128), _part_spec(16),
          _full_spec((128, 128)), _full_spec((NBASIS, 128, 128)),
          _full_spec((128, 64)), _full_spec((NBASIS, 128, 64)),
      ],
      out_specs=[_row_spec(128), _row_spec(64)],
      out_shape=[jax.ShapeDtypeStruct((N, 128), jnp.float32),
                 jax.ShapeDtypeStruct((N, 64), jnp.float32)],
  )(x, degp, wb1, ws1, wb2x, ws2x)


def _kan2_call(hs1, part1, pre2, degp, b1, wb2a, ws2a):
  return pl.pallas_call(
      _kan2_body,
      grid=(NBLK,),
      in_specs=[
          _row_spec(128), _part_spec(128), _row_spec(64), _part_spec(16),
          _full_spec((1, 128)),
          _full_spec((128, 64)), _full_spec((NBASIS, 128, 64)),
      ],
      out_specs=[_row_spec(32), _row_spec(32), _row_spec(16)],
      out_shape=[jax.ShapeDtypeStruct((N, 32), jnp.float32),
                 jax.ShapeDtypeStruct((N, 32), jnp.float32),
                 jax.ShapeDtypeStruct((N, 16), jnp.float32)],
  )(hs1, part1, pre2, degp, b1, wb2a, ws2a)


def _worker_id():
  return lax.axis_index("s") * NC + lax.axis_index("c")


def _sc_agg_kernel(f, nbuf):
  mesh = plsc.VectorSubcoreMesh(**_SC_MESH)

  @functools.partial(
      pl.kernel,
      out_type=jax.ShapeDtypeStruct((NC, NPAD, f), jnp.float32),
      mesh=mesh,
      scratch_types=[
          pltpu.VMEM((CHUNKS_PER_W, CHUNK), jnp.int32),
          pltpu.VMEM((CHUNKS_PER_W, CHUNK), jnp.int32),
          [pltpu.VMEM((CHUNK, f), jnp.float32)] * nbuf,
          pltpu.VMEM_SHARED((NPAD, f), jnp.float32),
          [pltpu.SemaphoreType.DMA] * nbuf,
          [pltpu.SemaphoreType.DMA] * nbuf,
      ],
      compiler_params=pltpu.CompilerParams(use_tc_tiling_on_sc=False),
  )
  def agg(rows_hbm, cols_hbm, hs_hbm, zeros_hbm, out_hbm,
          ridx, cidx, bufs, acc_sh, gsems, ssems):
    cid = lax.axis_index("c")
    sid = lax.axis_index("s")
    wid = _worker_id()
    rbase = sid * ROWS_PER_TILE
    cstart = wid * CHUNKS_PER_W
    pltpu.sync_copy(rows_hbm.at[pl.ds(cstart, CHUNKS_PER_W)], ridx)
    pltpu.sync_copy(cols_hbm.at[pl.ds(cstart, CHUNKS_PER_W)], cidx)
    pltpu.sync_copy(zeros_hbm.at[pl.ds(rbase, ROWS_PER_TILE)],
                    acc_sh.at[pl.ds(rbase, ROWS_PER_TILE)])
    plsc.subcore_barrier()

    def g_start(j, b):
      pltpu.async_copy(hs_hbm.at[ridx.at[j]], bufs[b], gsems[b])

    def g_wait(j, b):
      pltpu.make_async_copy(hs_hbm.at[ridx.at[j]], bufs[b], gsems[b]).wait()

    def s_start(j, b):
      pltpu.async_copy(bufs[b], acc_sh.at[cidx.at[j]], ssems[b], add=True)

    def s_wait(j, b):
      pltpu.make_async_copy(bufs[b], acc_sh.at[cidx.at[j]], ssems[b]).wait()

    for b in range(nbuf - 1):
      g_start(b, b)

    main = CHUNKS_PER_W // nbuf

    def body(i, carry):
      for b in range(nbuf):
        j = i * nbuf + b
        pj = j + nbuf - 1
        pb = (b + nbuf - 1) % nbuf

        if b == 0:
          @pl.when(i > 0)
          def _():
            s_wait(0, pb)
        else:
          s_wait(0, pb)

        @pl.when(pj < CHUNKS_PER_W)
        def _(pj=pj, pb=pb):
          g_start(pj, pb)

        g_wait(j, b)
        s_start(j, b)
      return carry

    lax.fori_loop(0, main, body, 0)
    for t in range(main * nbuf, CHUNKS_PER_W):
      g_wait(t, t % nbuf)
      s_start(t, t % nbuf)
    for c in range(main * nbuf - 1, CHUNKS_PER_W):
      s_wait(c, c % nbuf)
    plsc.subcore_barrier()
    pltpu.sync_copy(acc_sh.at[pl.ds(rbase, ROWS_PER_TILE)],
                    out_hbm.at[cid, pl.ds(rbase, ROWS_PER_TILE)])

  return agg


CPW2 = NCHUNKS // NS
LAST_ROWS = N - 15 * ROWS_PER_TILE


def _sc_agg_final(nbuf=4):
  mesh = plsc.VectorSubcoreMesh(**_SC_MESH)

  @functools.partial(
      pl.kernel,
      out_type=jax.ShapeDtypeStruct((N, 64), jnp.float32),
      mesh=mesh,
      scratch_types=[
          pltpu.VMEM((CPW2, CHUNK), jnp.int32),
          pltpu.VMEM((CPW2, CHUNK), jnp.int32),
          [pltpu.VMEM((CHUNK, 32), jnp.float32)] * nbuf,
          pltpu.VMEM((ROWS_PER_TILE, 32), jnp.float32),
          pltpu.VMEM((ROWS_PER_TILE, 32), jnp.float32),
          pltpu.VMEM((ROWS_PER_TILE, 16), jnp.float32),
          pltpu.VMEM((32,), jnp.float32),
          pltpu.VMEM_SHARED((NPAD, 32), jnp.float32),
          [pltpu.SemaphoreType.DMA] * nbuf,
          [pltpu.SemaphoreType.DMA] * nbuf,
      ],
      compiler_params=pltpu.CompilerParams(use_tc_tiling_on_sc=False),
  )
  def aggf(rows_hbm, cols_hbm, hs2a_hbm, hs2b_hbm, dinv_hbm, b2_hbm,
           zeros_hbm, out_hbm,
           ridx, cidx, bufs, accv, hsv, dinvv, b2v, acc_sh, gsems, ssems):
    cid = lax.axis_index("c")
    sid = lax.axis_index("s")
    rbase = sid * ROWS_PER_TILE
    cstart = sid * CPW2
    pltpu.sync_copy(rows_hbm.at[pl.ds(cstart, CPW2)], ridx)
    pltpu.sync_copy(cols_hbm.at[pl.ds(cstart, CPW2)], cidx)
    pltpu.sync_copy(zeros_hbm.at[pl.ds(rbase, ROWS_PER_TILE)],
                    acc_sh.at[pl.ds(rbase, ROWS_PER_TILE)])
    plsc.subcore_barrier()

    def g_start(j, b):
      @pl.when(cid == 0)
      def _():
        pltpu.async_copy(hs2a_hbm.at[ridx.at[j]], bufs[b], gsems[b])

      @pl.when(cid == 1)
      def _():
        pltpu.async_copy(hs2b_hbm.at[ridx.at[j]], bufs[b], gsems[b])

    def g_wait(j, b):
      pltpu.make_async_copy(hs2a_hbm.at[ridx.at[j]], bufs[b], gsems[b]).wait()

    def s_start(j, b):
      pltpu.async_copy(bufs[b], acc_sh.at[cidx.at[j]], ssems[b], add=True)

    def s_wait(j, b):
      pltpu.make_async_copy(bufs[b], acc_sh.at[cidx.at[j]], ssems[b]).wait()

    for b in range(nbuf - 1):
      g_start(b, b)

    main = CPW2 // nbuf

    def body(i, carry):
      for b in range(nbuf):
        j = i * nbuf + b
        pj = j + nbuf - 1
        pb = (b + nbuf - 1) % nbuf

        if b == 0:
          @pl.when(i > 0)
          def _():
            s_wait(0, pb)
        else:
          s_wait(0, pb)

        @pl.when(pj < CPW2)
        def _(pj=pj, pb=pb):
          g_start(pj, pb)

        g_wait(j, b)
        s_start(j, b)
      return carry

    lax.fori_loop(0, main, body, 0)
    for c in range(main * nbuf - 1, CPW2):
      s_wait(c, c % nbuf)
    plsc.subcore_barrier()

    nrows = jnp.where(sid == 15, LAST_ROWS, ROWS_PER_TILE)
    pltpu.sync_copy(acc_sh.at[pl.ds(rbase, ROWS_PER_TILE)], accv)
    pltpu.sync_copy(b2_hbm.at[cid], b2v)

    @pl.when(sid < 15)
    def _():
      @pl.when(cid == 0)
      def _():
        pltpu.sync_copy(hs2a_hbm.at[pl.ds(rbase, ROWS_PER_TILE)], hsv)

      @pl.when(cid == 1)
      def _():
        pltpu.sync_copy(hs2b_hbm.at[pl.ds(rbase, ROWS_PER_TILE)], hsv)
      pltpu.sync_copy(dinv_hbm.at[pl.ds(rbase, ROWS_PER_TILE)], dinvv)

    @pl.when(sid == 15)
    def _():
      @pl.when(cid == 0)
      def _():
        pltpu.sync_copy(hs2a_hbm.at[pl.ds(rbase, LAST_ROWS)],
                        hsv.at[pl.ds(0, LAST_ROWS)])

      @pl.when(cid == 1)
      def _():
        pltpu.sync_copy(hs2b_hbm.at[pl.ds(rbase, LAST_ROWS)],
                        hsv.at[pl.ds(0, LAST_ROWS)])
      pltpu.sync_copy(dinv_hbm.at[pl.ds(rbase, LAST_ROWS)],
                      dinvv.at[pl.ds(0, LAST_ROWS)])

    bias0 = b2v[pl.ds(0, 16)]
    bias1 = b2v[pl.ds(16, 16)]

    def erow(r, carry):
      dv = dinvv[r, pl.ds(0, 16)]
      a0 = accv[r, pl.ds(0, 16)]
      a1 = accv[r, pl.ds(16, 16)]
      h0 = hsv[r, pl.ds(0, 16)]
      h1 = hsv[r, pl.ds(16, 16)]
      accv[r, pl.ds(0, 16)] = dv * (a0 + h0) + bias0
      accv[r, pl.ds(16, 16)] = dv * (a1 + h1) + bias1
      return carry

    lax.fori_loop(0, nrows, erow, 0)

    @pl.when(sid < 15)
    def _():
      pltpu.sync_copy(accv,
                      out_hbm.at[pl.ds(rbase, ROWS_PER_TILE),
                                 pl.ds(cid * 32, 32)])

    @pl.when(sid == 15)
    def _():
      pltpu.sync_copy(accv.at[pl.ds(0, LAST_ROWS)],
                      out_hbm.at[pl.ds(rbase, LAST_ROWS),
                                 pl.ds(cid * 32, 32)])

  return aggf


def _sc_deg_kernel():
  mesh = plsc.VectorSubcoreMesh(**_SC_MESH)

  @functools.partial(
      pl.kernel,
      out_type=jax.ShapeDtypeStruct((NC, NPAD, 16), jnp.float32),
      mesh=mesh,
      scratch_types=[
          pltpu.VMEM((CHUNKS_PER_W, CHUNK), jnp.int32),
          pltpu.VMEM((CHUNK, 16), jnp.float32),
          pltpu.VMEM_SHARED((NPAD, 16), jnp.float32),
          pltpu.SemaphoreType.DMA,
      ],
      compiler_params=pltpu.CompilerParams(use_tc_tiling_on_sc=False),
  )
  def deg(cols_hbm, ones_hbm, zeros_hbm, out_hbm, cidx, ones_v, acc_sh, dsem):
    cid = lax.axis_index("c")
    sid = lax.axis_index("s")
    wid = _worker_id()
    rbase = sid * ROWS_PER_TILE
    cstart = wid * CHUNKS_PER_W
    pltpu.sync_copy(cols_hbm.at[pl.ds(cstart, CHUNKS_PER_W)], cidx)
    pltpu.sync_copy(ones_hbm, ones_v)
    pltpu.sync_copy(zeros_hbm.at[pl.ds(rbase, ROWS_PER_TILE)],
                    acc_sh.at[pl.ds(rbase, ROWS_PER_TILE)])
    plsc.subcore_barrier()

    WIN = 8

    def body(j, carry):
      @pl.when(j >= WIN)
      def _():
        pltpu.make_async_copy(ones_v, acc_sh.at[cidx.at[j]], dsem).wait()
      pltpu.async_copy(ones_v, acc_sh.at[cidx.at[j]], dsem, add=True)
      return carry

    lax.fori_loop(0, CHUNKS_PER_W, body, 0)

    def drain(j, carry):
      pltpu.make_async_copy(ones_v, acc_sh.at[cidx.at[j]], dsem).wait()
      return carry

    lax.fori_loop(0, WIN, drain, 0)
    plsc.subcore_barrier()
    pltpu.sync_copy(acc_sh.at[pl.ds(rbase, ROWS_PER_TILE)],
                    out_hbm.at[cid, pl.ds(rbase, ROWS_PER_TILE)])

  return deg


def kernel(x, edge_index, w1_base, w1_spline, w1_scaler, b1,
           w2_base, w2_spline, w2_scaler, b2):
  pad = EPAD - E
  pad_ar = jnp.arange(pad, dtype=jnp.int32)
  row = jnp.concatenate([edge_index[0], pad_ar % N])
  col = jnp.concatenate([edge_index[1], N + pad_ar % (NPAD - N)])
  rows_pk = row.reshape(NCHUNKS, CHUNK)
  cols_pk = col.reshape(NCHUNKS, CHUNK)

  ws1 = jnp.transpose(w1_spline * w1_scaler[:, :, None], (2, 1, 0))
  wb1 = w1_base.T
  ws2 = jnp.transpose(w2_spline * w2_scaler[:, :, None], (2, 1, 0))
  wb2 = w2_base.T
  wb2x, wb2a = wb2[:128], wb2[128:]
  ws2x, ws2a = ws2[:, :128, :], ws2[:, 128:, :]

  zeros16 = jnp.zeros((NPAD, 16), jnp.float32)
  zeros128 = jnp.zeros((NPAD, 128), jnp.float32)
  zeros32 = jnp.zeros((NPAD, 32), jnp.float32)
  ones16 = jnp.ones((CHUNK, 16), jnp.float32)

  degp = _sc_deg_kernel()(cols_pk, ones16, zeros16)
  hs1, pre2 = _kan1_call(x, degp, wb1, ws1, wb2x, ws2x)
  part1 = _sc_agg_kernel(128, 3)(rows_pk, cols_pk, hs1, zeros128)
  hs2a, hs2b, dinv16 = _kan2_call(hs1, part1, pre2, degp,
                                  b1.reshape(1, 128), wb2a, ws2a)
  return _sc_agg_final()(rows_pk, cols_pk, hs2a, hs2b, dinv16,
                         b2.reshape(2, 32), zeros32)


# speedup vs baseline: 1.0759x; 1.0759x over previous
import functools

import jax
import jax.numpy as jnp
from jax import lax
from jax.experimental import pallas as pl
from jax.experimental.pallas import tpu as pltpu
from jax.experimental.pallas import tpu_sc as plsc

GRID_SIZE = 4
SPLINE_ORDER = 3
NBASIS = GRID_SIZE + SPLINE_ORDER
N = 10000
E = 320000

NC, NS = 2, 16
NW = NC * NS
CHUNK = 64
NCHUNKS = 5120
CHUNKS_PER_W = NCHUNKS // NW
EPAD = NCHUNKS * CHUNK
ROWS_PER_TILE = 632
NPAD = NS * ROWS_PER_TILE
BLK = 1000
NBLK = N // BLK

_SC_MESH = dict(core_axis_name="c", subcore_axis_name="s",
                num_cores=NC, num_subcores=NS)


def _silu(x):
  return x * (1.0 / (1.0 + jnp.exp(-x)))


def _spline_bases(x):
  h = 2.0 / GRID_SIZE
  g = [(t - SPLINE_ORDER) * h - 1.0 for t in range(GRID_SIZE + 2 * SPLINE_ORDER + 1)]
  d = [x - g[t] for t in range(len(g))]
  b = [jnp.where((x >= g[t]) & (x < g[t + 1]), 1.0, 0.0).astype(x.dtype)
       for t in range(len(g) - 1)]
  for k in range(1, SPLINE_ORDER + 1):
    inv = 1.0 / (k * h)
    nb = []
    for t in range(len(b) - 1):
      left = (d[t] * inv) * b[t]
      right = (d[t + k + 1] * (-inv)) * b[t + 1]
      nb.append(left + right)
    b = nb
  return b


def _dinv_from_deg(deg_blk):
  degs = deg_blk[0] + deg_blk[1] + 1.0
  return lax.rsqrt(degs)[:, 0:1]


def _kan1_body(x_ref, deg_ref, wb1_ref, ws1_ref, wb2x_ref, ws2x_ref,
               hs1_ref, pre2_ref):
  xb = x_ref[:]
  dinv = _dinv_from_deg(deg_ref[:])
  sil = _silu(xb)
  bs = _spline_bases(xb)
  h1 = jnp.dot(sil, wb1_ref[:], preferred_element_type=jnp.float32)
  p2 = jnp.dot(sil, wb2x_ref[:], preferred_element_type=jnp.float32)
  for c in range(NBASIS):
    h1 = h1 + jnp.dot(bs[c], ws1_ref[c], preferred_element_type=jnp.float32)
    p2 = p2 + jnp.dot(bs[c], ws2x_ref[c], preferred_element_type=jnp.float32)
  hs1_ref[:] = h1 * dinv
  pre2_ref[:] = p2


def _kan2_body(hs1_ref, part_ref, pre2_ref, deg_ref, b1_ref, wb2a_ref,
               ws2a_ref, hs2_ref):
  dinv = _dinv_from_deg(deg_ref[:])
  agg = dinv * (part_ref[0] + part_ref[1] + hs1_ref[:]) + b1_ref[:]
  sil = _silu(agg)
  bs = _spline_bases(agg)
  h2 = pre2_ref[:] + jnp.dot(sil, wb2a_ref[:], preferred_element_type=jnp.float32)
  for c in range(NBASIS):
    h2 = h2 + jnp.dot(bs[c], ws2a_ref[c], preferred_element_type=jnp.float32)
  hs2_ref[:] = h2 * dinv


def _epilogue_body(hs2_ref, part_ref, deg_ref, b2_ref, out_ref):
  dinv = _dinv_from_deg(deg_ref[:])
  out_ref[:] = dinv * (part_ref[0] + part_ref[1] + hs2_ref[:]) + b2_ref[:]


def _row_spec(f):
  return pl.BlockSpec((BLK, f), lambda i: (i, 0))


def _part_spec(f):
  return pl.BlockSpec((2, BLK, f), lambda i: (0, i, 0))


def _full_spec(shape):
  nd = len(shape)
  return pl.BlockSpec(shape, lambda i, _nd=nd: (0,) * _nd)


def _kan1_call(x, degp, wb1, ws1, wb2x, ws2x):
  return pl.pallas_call(
      _kan1_body,
      grid=(NBLK,),
      in_specs=[
          _row_spec(128), _part_spec(16),
          _full_spec((128, 128)), _full_spec((NBASIS, 128, 128)),
          _full_spec((128, 64)), _full_spec((NBASIS, 128, 64)),
      ],
      out_specs=[_row_spec(128), _row_spec(64)],
      out_shape=[jax.ShapeDtypeStruct((N, 128), jnp.float32),
                 jax.ShapeDtypeStruct((N, 64), jnp.float32)],
  )(x, degp, wb1, ws1, wb2x, ws2x)


def _kan2_call(hs1, part1, pre2, degp, b1, wb2a, ws2a):
  return pl.pallas_call(
      _kan2_body,
      grid=(NBLK,),
      in_specs=[
          _row_spec(128), _part_spec(128), _row_spec(64), _part_spec(16),
          _full_spec((1, 128)),
          _full_spec((128, 64)), _full_spec((NBASIS, 128, 64)),
      ],
      out_specs=_row_spec(64),
      out_shape=jax.ShapeDtypeStruct((N, 64), jnp.float32),
  )(hs1, part1, pre2, degp, b1, wb2a, ws2a)


def _epilogue_call(hs2, part2, degp, b2):
  return pl.pallas_call(
      _epilogue_body,
      grid=(NBLK,),
      in_specs=[_row_spec(64), _part_spec(64), _part_spec(16),
                _full_spec((1, 64))],
      out_specs=_row_spec(64),
      out_shape=jax.ShapeDtypeStruct((N, 64), jnp.float32),
  )(hs2, part2, degp, b2)


def _worker_id():
  return lax.axis_index("s") * NC + lax.axis_index("c")


def _sc_agg_kernel(f, nbuf):
  mesh = plsc.VectorSubcoreMesh(**_SC_MESH)

  @functools.partial(
      pl.kernel,
      out_type=jax.ShapeDtypeStruct((NC, NPAD, f), jnp.float32),
      mesh=mesh,
      scratch_types=[
          pltpu.VMEM((CHUNKS_PER_W, CHUNK), jnp.int32),
          pltpu.VMEM((CHUNKS_PER_W, CHUNK), jnp.int32),
          [pltpu.VMEM((CHUNK, f), jnp.float32)] * nbuf,
          pltpu.VMEM_SHARED((NPAD, f), jnp.float32),
          [pltpu.SemaphoreType.DMA] * nbuf,
          [pltpu.SemaphoreType.DMA] * nbuf,
      ],
      compiler_params=pltpu.CompilerParams(use_tc_tiling_on_sc=False),
  )
  def agg(rows_hbm, cols_hbm, hs_hbm, zeros_hbm, out_hbm,
          ridx, cidx, bufs, acc_sh, gsems, ssems):
    cid = lax.axis_index("c")
    sid = lax.axis_index("s")
    wid = _worker_id()
    rbase = sid * ROWS_PER_TILE
    cstart = wid * CHUNKS_PER_W
    pltpu.sync_copy(rows_hbm.at[pl.ds(cstart, CHUNKS_PER_W)], ridx)
    pltpu.sync_copy(cols_hbm.at[pl.ds(cstart, CHUNKS_PER_W)], cidx)
    pltpu.sync_copy(zeros_hbm.at[pl.ds(rbase, ROWS_PER_TILE)],
                    acc_sh.at[pl.ds(rbase, ROWS_PER_TILE)])
    plsc.subcore_barrier()

    def g_start(j, b):
      pltpu.async_copy(hs_hbm.at[ridx.at[j]], bufs[b], gsems[b])

    def g_wait(j, b):
      pltpu.make_async_copy(hs_hbm.at[ridx.at[j]], bufs[b], gsems[b]).wait()

    def s_start(j, b):
      pltpu.async_copy(bufs[b], acc_sh.at[cidx.at[j]], ssems[b], add=True)

    def s_wait(j, b):
      pltpu.make_async_copy(bufs[b], acc_sh.at[cidx.at[j]], ssems[b]).wait()

    for b in range(nbuf - 1):
      g_start(b, b)

    main = CHUNKS_PER_W // nbuf

    def body(i, carry):
      for b in range(nbuf):
        j = i * nbuf + b
        pj = j + nbuf - 1
        pb = (b + nbuf - 1) % nbuf

        if b == 0:
          @pl.when(i > 0)
          def _():
            s_wait(0, pb)
        else:
          s_wait(0, pb)

        @pl.when(pj < CHUNKS_PER_W)
        def _(pj=pj, pb=pb):
          g_start(pj, pb)

        g_wait(j, b)
        s_start(j, b)
      return carry

    lax.fori_loop(0, main, body, 0)
    for t in range(main * nbuf, CHUNKS_PER_W):
      g_wait(t, t % nbuf)
      s_start(t, t % nbuf)
    for c in range(main * nbuf - 1, CHUNKS_PER_W):
      s_wait(c, c % nbuf)
    plsc.subcore_barrier()
    pltpu.sync_copy(acc_sh.at[pl.ds(rbase, ROWS_PER_TILE)],
                    out_hbm.at[cid, pl.ds(rbase, ROWS_PER_TILE)])

  return agg


def _sc_deg_kernel():
  mesh = plsc.VectorSubcoreMesh(**_SC_MESH)

  @functools.partial(
      pl.kernel,
      out_type=jax.ShapeDtypeStruct((NC, NPAD, 16), jnp.float32),
      mesh=mesh,
      scratch_types=[
          pltpu.VMEM((CHUNKS_PER_W, CHUNK), jnp.int32),
          pltpu.VMEM((CHUNK, 16), jnp.float32),
          pltpu.VMEM_SHARED((NPAD, 16), jnp.float32),
          pltpu.SemaphoreType.DMA,
      ],
      compiler_params=pltpu.CompilerParams(use_tc_tiling_on_sc=False),
  )
  def deg(cols_hbm, ones_hbm, zeros_hbm, out_hbm, cidx, ones_v, acc_sh, dsem):
    cid = lax.axis_index("c")
    sid = lax.axis_index("s")
    wid = _worker_id()
    rbase = sid * ROWS_PER_TILE
    cstart = wid * CHUNKS_PER_W
    pltpu.sync_copy(cols_hbm.at[pl.ds(cstart, CHUNKS_PER_W)], cidx)
    pltpu.sync_copy(ones_hbm, ones_v)
    pltpu.sync_copy(zeros_hbm.at[pl.ds(rbase, ROWS_PER_TILE)],
                    acc_sh.at[pl.ds(rbase, ROWS_PER_TILE)])
    plsc.subcore_barrier()

    WIN = 8

    def body(j, carry):
      @pl.when(j >= WIN)
      def _():
        pltpu.make_async_copy(ones_v, acc_sh.at[cidx.at[j]], dsem).wait()
      pltpu.async_copy(ones_v, acc_sh.at[cidx.at[j]], dsem, add=True)
      return carry

    lax.fori_loop(0, CHUNKS_PER_W, body, 0)

    def drain(j, carry):
      pltpu.make_async_copy(ones_v, acc_sh.at[cidx.at[j]], dsem).wait()
      return carry

    lax.fori_loop(0, WIN, drain, 0)
    plsc.subcore_barrier()
    pltpu.sync_copy(acc_sh.at[pl.ds(rbase, ROWS_PER_TILE)],
                    out_hbm.at[cid, pl.ds(rbase, ROWS_PER_TILE)])

  return deg


def kernel(x, edge_index, w1_base, w1_spline, w1_scaler, b1,
           w2_base, w2_spline, w2_scaler, b2):
  pad = EPAD - E
  pad_ar = jnp.arange(pad, dtype=jnp.int32)
  row = jnp.concatenate([edge_index[0], pad_ar % N])
  col = jnp.concatenate([edge_index[1], N + pad_ar % (NPAD - N)])
  rows_pk = row.reshape(NCHUNKS, CHUNK)
  cols_pk = col.reshape(NCHUNKS, CHUNK)

  ws1 = jnp.transpose(w1_spline * w1_scaler[:, :, None], (2, 1, 0))
  wb1 = w1_base.T
  ws2 = jnp.transpose(w2_spline * w2_scaler[:, :, None], (2, 1, 0))
  wb2 = w2_base.T
  wb2x, wb2a = wb2[:128], wb2[128:]
  ws2x, ws2a = ws2[:, :128, :], ws2[:, 128:, :]

  zeros16 = jnp.zeros((NPAD, 16), jnp.float32)
  zeros128 = jnp.zeros((NPAD, 128), jnp.float32)
  zeros64 = jnp.zeros((NPAD, 64), jnp.float32)
  ones16 = jnp.ones((CHUNK, 16), jnp.float32)

  degp = _sc_deg_kernel()(cols_pk, ones16, zeros16)
  hs1, pre2 = _kan1_call(x, degp, wb1, ws1, wb2x, ws2x)
  part1 = _sc_agg_kernel(128, 3)(rows_pk, cols_pk, hs1, zeros128)
  hs2 = _kan2_call(hs1, part1, pre2, degp, b1.reshape(1, 128), wb2a, ws2a)
  part2 = _sc_agg_kernel(64, 4)(rows_pk, cols_pk, hs2, zeros64)
  return _epilogue_call(hs2, part2, degp, b2.reshape(1, 64))


# speedup vs baseline: 1.0861x; 1.0095x over previous
import functools

import jax
import jax.numpy as jnp
from jax import lax
from jax.experimental import pallas as pl
from jax.experimental.pallas import tpu as pltpu
from jax.experimental.pallas import tpu_sc as plsc

GRID_SIZE = 4
SPLINE_ORDER = 3
NBASIS = GRID_SIZE + SPLINE_ORDER
N = 10000
E = 320000

NC, NS = 2, 16
NW = NC * NS
CHUNK = 64
NCHUNKS = 5120
CHUNKS_PER_W = NCHUNKS // NW
EPAD = NCHUNKS * CHUNK
ROWS_PER_TILE = 632
NPAD = NS * ROWS_PER_TILE
BLK = 2000
NBLK = N // BLK

_SC_MESH = dict(core_axis_name="c", subcore_axis_name="s",
                num_cores=NC, num_subcores=NS)


def _silu(x):
  return x * (1.0 / (1.0 + jnp.exp(-x)))


def _spline_bases(x):
  h = 2.0 / GRID_SIZE
  g = [(t - SPLINE_ORDER) * h - 1.0 for t in range(GRID_SIZE + 2 * SPLINE_ORDER + 1)]
  d = [x - g[t] for t in range(len(g))]
  b = [jnp.where((x >= g[t]) & (x < g[t + 1]), 1.0, 0.0).astype(x.dtype)
       for t in range(len(g) - 1)]
  for k in range(1, SPLINE_ORDER + 1):
    inv = 1.0 / (k * h)
    nb = []
    for t in range(len(b) - 1):
      left = (d[t] * inv) * b[t]
      right = (d[t + k + 1] * (-inv)) * b[t + 1]
      nb.append(left + right)
    b = nb
  return b


def _dinv_from_deg(deg_blk):
  degs = deg_blk[0] + deg_blk[1] + 1.0
  return lax.rsqrt(degs)[:, 0:1]


def _kan1_body(x_ref, deg_ref, wb1_ref, ws1_ref, wb2x_ref, ws2x_ref,
               hs1_ref, pre2_ref):
  xb = x_ref[:]
  dinv = _dinv_from_deg(deg_ref[:])
  sil = _silu(xb)
  bs = _spline_bases(xb)
  h1 = jnp.dot(sil, wb1_ref[:], preferred_element_type=jnp.float32)
  p2 = jnp.dot(sil, wb2x_ref[:], preferred_element_type=jnp.float32)
  for c in range(NBASIS):
    h1 = h1 + jnp.dot(bs[c], ws1_ref[c], preferred_element_type=jnp.float32)
    p2 = p2 + jnp.dot(bs[c], ws2x_ref[c], preferred_element_type=jnp.float32)
  hs1_ref[:] = h1 * dinv
  pre2_ref[:] = p2


def _kan2_body(hs1_ref, part_ref, pre2_ref, deg_ref, b1_ref, wb2a_ref,
               ws2a_ref, hs2_ref):
  dinv = _dinv_from_deg(deg_ref[:])
  agg = dinv * (part_ref[0] + part_ref[1] + hs1_ref[:]) + b1_ref[:]
  sil = _silu(agg)
  bs = _spline_bases(agg)
  h2 = pre2_ref[:] + jnp.dot(sil, wb2a_ref[:], preferred_element_type=jnp.float32)
  for c in range(NBASIS):
    h2 = h2 + jnp.dot(bs[c], ws2a_ref[c], preferred_element_type=jnp.float32)
  hs2_ref[:] = h2 * dinv


def _epilogue_body(hs2_ref, part_ref, deg_ref, b2_ref, out_ref):
  dinv = _dinv_from_deg(deg_ref[:])
  out_ref[:] = dinv * (part_ref[0] + part_ref[1] + hs2_ref[:]) + b2_ref[:]


def _row_spec(f):
  return pl.BlockSpec((BLK, f), lambda i: (i, 0))


def _part_spec(f):
  return pl.BlockSpec((2, BLK, f), lambda i: (0, i, 0))


def _full_spec(shape):
  nd = len(shape)
  return pl.BlockSpec(shape, lambda i, _nd=nd: (0,) * _nd)


def _kan1_call(x, degp, wb1, ws1, wb2x, ws2x):
  return pl.pallas_call(
      _kan1_body,
      grid=(NBLK,),
      in_specs=[
          _row_spec(128), _part_spec(16),
          _full_spec((128, 128)), _full_spec((NBASIS, 128, 128)),
          _full_spec((128, 64)), _full_spec((NBASIS, 128, 64)),
      ],
      out_specs=[_row_spec(128), _row_spec(64)],
      out_shape=[jax.ShapeDtypeStruct((N, 128), jnp.float32),
                 jax.ShapeDtypeStruct((N, 64), jnp.float32)],
  )(x, degp, wb1, ws1, wb2x, ws2x)


def _kan2_call(hs1, part1, pre2, degp, b1, wb2a, ws2a):
  return pl.pallas_call(
      _kan2_body,
      grid=(NBLK,),
      in_specs=[
          _row_spec(128), _part_spec(128), _row_spec(64), _part_spec(16),
          _full_spec((1, 128)),
          _full_spec((128, 64)), _full_spec((NBASIS, 128, 64)),
      ],
      out_specs=_row_spec(64),
      out_shape=jax.ShapeDtypeStruct((N, 64), jnp.float32),
  )(hs1, part1, pre2, degp, b1, wb2a, ws2a)


def _epilogue_call(hs2, part2, degp, b2):
  return pl.pallas_call(
      _epilogue_body,
      grid=(NBLK,),
      in_specs=[_row_spec(64), _part_spec(64), _part_spec(16),
                _full_spec((1, 64))],
      out_specs=_row_spec(64),
      out_shape=jax.ShapeDtypeStruct((N, 64), jnp.float32),
  )(hs2, part2, degp, b2)


def _worker_id():
  return lax.axis_index("s") * NC + lax.axis_index("c")


def _sc_agg_kernel(f, nbuf):
  mesh = plsc.VectorSubcoreMesh(**_SC_MESH)

  @functools.partial(
      pl.kernel,
      out_type=jax.ShapeDtypeStruct((NC, NPAD, f), jnp.float32),
      mesh=mesh,
      scratch_types=[
          pltpu.VMEM((CHUNKS_PER_W, CHUNK), jnp.int32),
          pltpu.VMEM((CHUNKS_PER_W, CHUNK), jnp.int32),
          [pltpu.VMEM((CHUNK, f), jnp.float32)] * nbuf,
          pltpu.VMEM_SHARED((NPAD, f), jnp.float32),
          [pltpu.SemaphoreType.DMA] * nbuf,
          [pltpu.SemaphoreType.DMA] * nbuf,
      ],
      compiler_params=pltpu.CompilerParams(use_tc_tiling_on_sc=False),
  )
  def agg(rows_hbm, cols_hbm, hs_hbm, zeros_hbm, out_hbm,
          ridx, cidx, bufs, acc_sh, gsems, ssems):
    cid = lax.axis_index("c")
    sid = lax.axis_index("s")
    wid = _worker_id()
    rbase = sid * ROWS_PER_TILE
    cstart = wid * CHUNKS_PER_W
    pltpu.sync_copy(rows_hbm.at[pl.ds(cstart, CHUNKS_PER_W)], ridx)
    pltpu.sync_copy(cols_hbm.at[pl.ds(cstart, CHUNKS_PER_W)], cidx)
    pltpu.sync_copy(zeros_hbm.at[pl.ds(rbase, ROWS_PER_TILE)],
                    acc_sh.at[pl.ds(rbase, ROWS_PER_TILE)])
    plsc.subcore_barrier()

    def g_start(j, b):
      pltpu.async_copy(hs_hbm.at[ridx.at[j]], bufs[b], gsems[b])

    def g_wait(j, b):
      pltpu.make_async_copy(hs_hbm.at[ridx.at[j]], bufs[b], gsems[b]).wait()

    def s_start(j, b):
      pltpu.async_copy(bufs[b], acc_sh.at[cidx.at[j]], ssems[b], add=True)

    def s_wait(j, b):
      pltpu.make_async_copy(bufs[b], acc_sh.at[cidx.at[j]], ssems[b]).wait()

    for b in range(nbuf - 1):
      g_start(b, b)

    main = CHUNKS_PER_W // nbuf

    def body(i, carry):
      for b in range(nbuf):
        j = i * nbuf + b
        pj = j + nbuf - 1
        pb = (b + nbuf - 1) % nbuf

        if b == 0:
          @pl.when(i > 0)
          def _():
            s_wait(0, pb)
        else:
          s_wait(0, pb)

        @pl.when(pj < CHUNKS_PER_W)
        def _(pj=pj, pb=pb):
          g_start(pj, pb)

        g_wait(j, b)
        s_start(j, b)
      return carry

    lax.fori_loop(0, main, body, 0)
    for t in range(main * nbuf, CHUNKS_PER_W):
      g_wait(t, t % nbuf)
      s_start(t, t % nbuf)
    for c in range(main * nbuf - 1, CHUNKS_PER_W):
      s_wait(c, c % nbuf)
    plsc.subcore_barrier()
    pltpu.sync_copy(acc_sh.at[pl.ds(rbase, ROWS_PER_TILE)],
                    out_hbm.at[cid, pl.ds(rbase, ROWS_PER_TILE)])

  return agg


def _sc_deg_kernel():
  mesh = plsc.VectorSubcoreMesh(**_SC_MESH)

  @functools.partial(
      pl.kernel,
      out_type=jax.ShapeDtypeStruct((NC, NPAD, 16), jnp.float32),
      mesh=mesh,
      scratch_types=[
          pltpu.VMEM((CHUNKS_PER_W, CHUNK), jnp.int32),
          pltpu.VMEM((CHUNK, 16), jnp.float32),
          pltpu.VMEM_SHARED((NPAD, 16), jnp.float32),
          pltpu.SemaphoreType.DMA,
      ],
      compiler_params=pltpu.CompilerParams(use_tc_tiling_on_sc=False),
  )
  def deg(cols_hbm, ones_hbm, zeros_hbm, out_hbm, cidx, ones_v, acc_sh, dsem):
    cid = lax.axis_index("c")
    sid = lax.axis_index("s")
    wid = _worker_id()
    rbase = sid * ROWS_PER_TILE
    cstart = wid * CHUNKS_PER_W
    pltpu.sync_copy(cols_hbm.at[pl.ds(cstart, CHUNKS_PER_W)], cidx)
    pltpu.sync_copy(ones_hbm, ones_v)
    pltpu.sync_copy(zeros_hbm.at[pl.ds(rbase, ROWS_PER_TILE)],
                    acc_sh.at[pl.ds(rbase, ROWS_PER_TILE)])
    plsc.subcore_barrier()

    WIN = 8

    def body(j, carry):
      @pl.when(j >= WIN)
      def _():
        pltpu.make_async_copy(ones_v, acc_sh.at[cidx.at[j]], dsem).wait()
      pltpu.async_copy(ones_v, acc_sh.at[cidx.at[j]], dsem, add=True)
      return carry

    lax.fori_loop(0, CHUNKS_PER_W, body, 0)

    def drain(j, carry):
      pltpu.make_async_copy(ones_v, acc_sh.at[cidx.at[j]], dsem).wait()
      return carry

    lax.fori_loop(0, WIN, drain, 0)
    plsc.subcore_barrier()
    pltpu.sync_copy(acc_sh.at[pl.ds(rbase, ROWS_PER_TILE)],
                    out_hbm.at[cid, pl.ds(rbase, ROWS_PER_TILE)])

  return deg


def kernel(x, edge_index, w1_base, w1_spline, w1_scaler, b1,
           w2_base, w2_spline, w2_scaler, b2):
  pad = EPAD - E
  pad_ar = jnp.arange(pad, dtype=jnp.int32)
  row = jnp.concatenate([edge_index[0], pad_ar % N])
  col = jnp.concatenate([edge_index[1], N + pad_ar % (NPAD - N)])
  rows_pk = row.reshape(NCHUNKS, CHUNK)
  cols_pk = col.reshape(NCHUNKS, CHUNK)

  ws1 = jnp.transpose(w1_spline * w1_scaler[:, :, None], (2, 1, 0))
  wb1 = w1_base.T
  ws2 = jnp.transpose(w2_spline * w2_scaler[:, :, None], (2, 1, 0))
  wb2 = w2_base.T
  wb2x, wb2a = wb2[:128], wb2[128:]
  ws2x, ws2a = ws2[:, :128, :], ws2[:, 128:, :]

  zeros16 = jnp.zeros((NPAD, 16), jnp.float32)
  zeros128 = jnp.zeros((NPAD, 128), jnp.float32)
  zeros64 = jnp.zeros((NPAD, 64), jnp.float32)
  ones16 = jnp.ones((CHUNK, 16), jnp.float32)

  degp = _sc_deg_kernel()(cols_pk, ones16, zeros16)
  hs1, pre2 = _kan1_call(x, degp, wb1, ws1, wb2x, ws2x)
  part1 = _sc_agg_kernel(128, 3)(rows_pk, cols_pk, hs1, zeros128)
  hs2 = _kan2_call(hs1, part1, pre2, degp, b1.reshape(1, 128), wb2a, ws2a)
  part2 = _sc_agg_kernel(64, 4)(rows_pk, cols_pk, hs2, zeros64)
  return _epilogue_call(hs2, part2, degp, b2.reshape(1, 64))
